# Initial kernel scaffold; baseline (speedup 1.0000x reference)
import jax
import jax.numpy as jnp
from jax.experimental import pallas as pl


def kernel(hidden_states, W_A, gate_scale, gate_bias, W_gate, W_up, W_down):
    raise NotImplementedError("write your pallas kernel here")


# speedup vs baseline: 1.1299x; 1.1299x over previous
import functools

import jax
import jax.numpy as jnp
from jax.experimental import pallas as pl
from jax.experimental.pallas import tpu as pltpu

_THRESHOLD = 0.5


def _gate_kernel(x_ref, wa_ref, m_ref, scale_ref, bias_ref, gout_ref, gw_ref):
    x = x_ref[...].astype(jnp.bfloat16)
    gh = jax.lax.dot_general(
        x, wa_ref[...].astype(jnp.bfloat16), (((1,), (1,)), ((), ())),
        preferred_element_type=jnp.float32)
    g2 = gh * gh
    s2 = jax.lax.dot_general(
        g2, m_ref[...], (((1,), (0,)), ((), ())),
        precision=jax.lax.Precision.HIGHEST,
        preferred_element_type=jnp.float32)
    scores = jnp.sqrt(s2 + 1e-6) * scale_ref[...] - bias_ref[...]
    mask = scores >= _THRESHOLD
    gout_ref[...] = jnp.where(mask, scores, -jnp.inf)
    gw_ref[...] = jnp.where(mask, scores, 0.0)


def _ffn_kernel(x_ref, gw_ref, wg_ref, wu_ref, wd_ref, out_ref, *, half):
    e = pl.program_id(0)
    f = pl.program_id(1)

    @pl.when((f == 0) & (e % half == 0))
    def _init():
        out_ref[...] = jnp.zeros_like(out_ref)

    x = x_ref[...]
    wg = wg_ref[0]
    wu = wu_ref[0]
    wd = wd_ref[0]
    xg = jax.lax.dot_general(x, wg, (((1,), (1,)), ((), ())),
                             preferred_element_type=jnp.float32)
    xu = jax.lax.dot_general(x, wu, (((1,), (1,)), ((), ())),
                             preferred_element_type=jnp.float32)
    h = xg * jax.nn.sigmoid(xg) * xu
    gw = gw_ref[...]
    lane = jax.lax.broadcasted_iota(jnp.int32, gw.shape, 1)
    gcol = jnp.sum(jnp.where(lane == e, gw, 0.0), axis=1, keepdims=True)
    hs = (h * gcol).astype(jnp.bfloat16)
    contrib = jax.lax.dot_general(hs, wd, (((1,), (1,)), ((), ())),
                                  preferred_element_type=jnp.float32)
    out_ref[0] += contrib


def kernel(hidden_states, W_A, gate_scale, gate_bias, W_gate, W_up, W_down):
    orig_shape = hidden_states.shape
    D = orig_shape[-1]
    x = hidden_states.reshape(-1, D)
    N = x.shape[0]
    E, R, _ = W_A.shape
    DFF = W_gate.shape[1]
    FB = 256
    F = DFF // FB

    wa2 = W_A.reshape(E * R, D)
    m = jnp.repeat(jnp.eye(E, dtype=jnp.float32), R, axis=0) / R
    TGB = 512
    gate_out, gw = pl.pallas_call(
        _gate_kernel,
        grid=(N // TGB,),
        in_specs=[
            pl.BlockSpec((TGB, D), lambda t: (t, 0)),
            pl.BlockSpec((E * R, D), lambda t: (0, 0)),
            pl.BlockSpec((E * R, E), lambda t: (0, 0)),
            pl.BlockSpec((1, E), lambda t: (0, 0)),
            pl.BlockSpec((1, E), lambda t: (0, 0)),
        ],
        out_specs=[
            pl.BlockSpec((TGB, E), lambda t: (t, 0)),
            pl.BlockSpec((TGB, E), lambda t: (t, 0)),
        ],
        out_shape=[jax.ShapeDtypeStruct((N, E), jnp.float32)] * 2,
    )(x, wa2, m, gate_scale.reshape(1, E), gate_bias.reshape(1, E))

    xb = x.astype(jnp.bfloat16)
    wgb = W_gate.astype(jnp.bfloat16)
    wub = W_up.astype(jnp.bfloat16)
    wdb = W_down.astype(jnp.bfloat16)
    half = E // 2
    out2 = pl.pallas_call(
        functools.partial(_ffn_kernel, half=half),
        grid=(E, F),
        in_specs=[
            pl.BlockSpec((N, D), lambda e, f: (0, 0)),
            pl.BlockSpec((N, E), lambda e, f: (0, 0)),
            pl.BlockSpec((1, FB, D), lambda e, f: (e, f, 0)),
            pl.BlockSpec((1, FB, D), lambda e, f: (e, f, 0)),
            pl.BlockSpec((1, D, FB), lambda e, f: (e, 0, f)),
        ],
        out_specs=pl.BlockSpec((1, N, D), lambda e, f: (e // half, 0, 0)),
        out_shape=jax.ShapeDtypeStruct((2, N, D), jnp.float32),
        compiler_params=pltpu.CompilerParams(
            dimension_semantics=("parallel", "arbitrary")),
    )(xb, gw, wgb, wub, wdb)
    out = (out2[0] + out2[1]).reshape(orig_shape)
    return out, gate_out.reshape(orig_shape[:-1] + (E,))
